# Initial kernel scaffold; baseline (speedup 1.0000x reference)
import jax
import jax.numpy as jnp
from jax.experimental import pallas as pl


def kernel(x, edge_index, edge_attr, batch, params):
    raise NotImplementedError("write your pallas kernel here")


# speedup vs baseline: 3.9919x; 3.9919x over previous
import functools

import jax
import jax.numpy as jnp
from jax import lax
from jax.experimental import pallas as pl
from jax.experimental.pallas import tpu as pltpu
from jax.experimental.pallas import tpu_sc as plsc

N = 10000
E = 320000
D = 128
G = 16

NC = 2
NS = 16
NW = NC * NS
L = 16

NPT = 320
NPAD = NW * NPT

NCH_SCAN = 40
SCAN_CH = E // NCH_SCAN
DCH = 8192
CAP = 40 * DCH
GCH = 128

BR = 1024

_mesh = plsc.VectorSubcoreMesh(core_axis_name="c", subcore_axis_name="s")


def _worker_id():
    return lax.axis_index("s") * NC + lax.axis_index("c")


@functools.partial(
    pl.kernel,
    out_type=(
        jax.ShapeDtypeStruct((NW * CAP,), jnp.int32),
        jax.ShapeDtypeStruct((NW * CAP,), jnp.float32),
        jax.ShapeDtypeStruct((NW * 16,), jnp.int32),
        jax.ShapeDtypeStruct((NPAD,), jnp.float32),
    ),
    mesh=_mesh,
    scratch_types=[
        pltpu.VMEM((DCH,), jnp.int32),
        pltpu.VMEM((DCH,), jnp.int32),
        pltpu.VMEM((DCH,), jnp.float32),
        pltpu.VMEM((SCAN_CH + 16,), jnp.int32),
        pltpu.VMEM((SCAN_CH + 16,), jnp.float32),
        pltpu.VMEM((NPT * 16,), jnp.float32),
        pltpu.VMEM((NPT,), jnp.float32),
        pltpu.VMEM((16,), jnp.int32),
    ],
    compiler_params=pltpu.CompilerParams(needs_layout_passes=False),
)
def _bucket(esrc, edst, ea, pk_out, w_out, cnt_out, deg_out,
            srcb, dstb, wb, stg_pk, stg_w, degacc, degv, cnt_stg):
    wid = _worker_id()
    lo = wid * NPT
    iota = lax.iota(jnp.int32, 16)
    z16i = jnp.zeros((16,), jnp.int32)
    z16f = jnp.zeros((16,), jnp.float32)

    def chunk_body(c, gtot):
        gtot = pl.multiple_of(gtot, 16)
        base = pl.multiple_of(c * SCAN_CH, 16)
        pltpu.sync_copy(esrc.at[pl.ds(base, SCAN_CH)], srcb.at[pl.ds(0, SCAN_CH)])
        pltpu.sync_copy(edst.at[pl.ds(base, SCAN_CH)], dstb.at[pl.ds(0, SCAN_CH)])
        pltpu.sync_copy(ea.at[pl.ds(base, SCAN_CH)], wb.at[pl.ds(0, SCAN_CH)])

        def grp(k, off):
            sl = pl.ds(k * 16, 16)
            dv = dstb[sl]
            m = (dv >= lo) & (dv < lo + NPT)
            pk = ((dv - lo) << 16) | srcb[sl]
            csum = plsc.cumsum(m.astype(jnp.int32))
            idx = off + csum - 1
            plsc.store_scatter(stg_pk, [idx], pk, mask=m)
            plsc.store_scatter(stg_w, [idx], wb[sl], mask=m)
            return off + csum[15]

        cnt_c = lax.fori_loop(0, SCAN_CH // 16, grp, 0)
        stg_pk[pl.ds(cnt_c, 16)] = z16i
        stg_w[pl.ds(cnt_c, 16)] = z16f
        cnt_r = (cnt_c + 15) & ~15
        off = pl.multiple_of(wid * CAP + gtot, 16)
        pltpu.sync_copy(stg_pk.at[pl.ds(0, SCAN_CH)],
                        pk_out.at[pl.ds(off, SCAN_CH)])
        pltpu.sync_copy(stg_w.at[pl.ds(0, SCAN_CH)],
                        w_out.at[pl.ds(off, SCAN_CH)])
        return gtot + cnt_r

    total = lax.fori_loop(0, NCH_SCAN, chunk_body, 0)
    cnt_stg[...] = jnp.full((16,), total, jnp.int32)
    pltpu.sync_copy(cnt_stg, cnt_out.at[pl.ds(pl.multiple_of(wid * 16, 16), 16)])

    def zro(k, _):
        degacc[pl.ds(k * 16, 16)] = z16f
        return 0

    lax.fori_loop(0, NPT, zro, 0)

    nch = (total + DCH - 1) >> 13

    def dchunk(c, _):
        base = c * DCH
        off = pl.multiple_of(wid * CAP + base, 16)
        pltpu.sync_copy(pk_out.at[pl.ds(off, DCH)], srcb)
        pltpu.sync_copy(w_out.at[pl.ds(off, DCH)], wb)

        def g2(k, _):
            sl = pl.ds(k * 16, 16)
            valid = (base + k * 16 + iota) < total
            w16 = jnp.where(valid, wb[sl], 0.0)
            pk16 = jnp.where(valid, srcb[sl], 0)
            dl16 = lax.shift_right_logical(pk16, 16)
            for i in range(16):
                plsc.addupdate(degacc.at[pl.ds(dl16[i] * 16, 16)],
                               jnp.full((16,), w16[i]))
            return 0

        lax.fori_loop(0, DCH // 16, g2, 0)
        return 0

    lax.fori_loop(0, nch, dchunk, 0)

    def ext(k, _):
        idx = (k * 16 + iota) * 16
        degv[pl.ds(k * 16, 16)] = plsc.load_gather(degacc, [idx])
        return 0

    lax.fori_loop(0, NPT // 16, ext, 0)
    pltpu.sync_copy(degv, deg_out.at[pl.ds(pl.multiple_of(wid * NPT, 16), NPT)])


@functools.partial(
    pl.kernel,
    out_type=jax.ShapeDtypeStruct((NPAD, D), jnp.float32),
    mesh=_mesh,
    scratch_types=[
        pltpu.VMEM((NPAD,), jnp.float32),
        pltpu.VMEM((16,), jnp.int32),
        pltpu.VMEM((GCH,), jnp.int32),
        pltpu.VMEM((GCH,), jnp.float32),
        pltpu.VMEM((GCH,), jnp.int32),
        pltpu.VMEM((GCH,), jnp.float32),
        pltpu.VMEM((GCH, D), jnp.float32),
        pltpu.VMEM((NPT, D), jnp.float32),
        pltpu.VMEM((NPT, D), jnp.float32),
    ],
    compiler_params=pltpu.CompilerParams(needs_layout_passes=False),
)
def _scatter(xw, dinv, pk_in, w_in, cnt_in, out,
             dinv_v, cnt_v, pkb, wb, srcv, fv, rows, acc, xwown):
    wid = _worker_id()
    lo = wid * NPT
    iota = lax.iota(jnp.int32, 16)
    z16f = jnp.zeros((16,), jnp.float32)

    pltpu.sync_copy(dinv, dinv_v)
    pltpu.sync_copy(cnt_in.at[pl.ds(pl.multiple_of(wid * 16, 16), 16)], cnt_v)
    total = cnt_v[...][0]

    def zro(r, _):
        for j in range(D // 16):
            acc[r, pl.ds(j * 16, 16)] = z16f
        return 0

    lax.fori_loop(0, NPT, zro, 0)

    nch = (total + GCH - 1) >> 7

    def chunk(c, _):
        base = c * GCH
        off = pl.multiple_of(wid * CAP + base, 16)
        pltpu.sync_copy(pk_in.at[pl.ds(off, GCH)], pkb)
        pltpu.sync_copy(w_in.at[pl.ds(off, GCH)], wb)

        def p1(k, _):
            sl = pl.ds(k * 16, 16)
            valid = (base + k * 16 + iota) < total
            pk16 = jnp.where(valid, pkb[sl], 0)
            src16 = pk16 & 0xFFFF
            f16 = jnp.where(valid, wb[sl], 0.0) * plsc.load_gather(dinv_v, [src16])
            pkb[sl] = pk16
            srcv[sl] = src16
            fv[sl] = f16
            return 0

        lax.fori_loop(0, GCH // 16, p1, 0)
        pltpu.sync_copy(xw.at[srcv], rows)

        def p2(k, _):
            sl = pl.ds(k * 16, 16)
            f16 = fv[sl]
            dl16 = lax.shift_right_logical(pkb[sl], 16)
            for i in range(16):
                fs = f16[i]
                dl = dl16[i]
                e = k * 16 + i
                for j in range(D // 16):
                    cs = pl.ds(j * 16, 16)
                    plsc.addupdate(acc.at[dl, cs], fs * rows[e, cs])
            return 0

        lax.fori_loop(0, GCH // 16, p2, 0)
        return 0

    lax.fori_loop(0, nch, chunk, 0)

    pltpu.sync_copy(xw.at[pl.ds(lo, NPT)], xwown)

    def ep(k, _):
        d16 = dinv_v[pl.ds(lo + k * 16, 16)]
        for i in range(16):
            dsc = d16[i]
            r = k * 16 + i
            for j in range(D // 16):
                cs = pl.ds(j * 16, 16)
                acc[r, cs] = dsc * (acc[r, cs] + dsc * xwown[r, cs])
        return 0

    lax.fori_loop(0, NPT // 16, ep, 0)
    pltpu.sync_copy(acc, out.at[pl.ds(lo, NPT)])


def _dinv_body(deg_ref, o_ref):
    d = deg_ref[...] + 1.0
    o_ref[...] = jnp.where(d > 0.0, lax.rsqrt(jnp.maximum(d, 1e-12)), 0.0)


def _dinv(deg):
    deg2 = deg.reshape(NPAD // D, D)
    o = pl.pallas_call(
        _dinv_body,
        out_shape=jax.ShapeDtypeStruct((NPAD // D, D), jnp.float32),
    )(deg2)
    return o.reshape(NPAD)


def _mm_body(h_ref, w_ref, o_ref):
    o_ref[...] = jnp.dot(h_ref[...], w_ref[...],
                         preferred_element_type=jnp.float32)


def _mm0(h, w):
    return pl.pallas_call(
        _mm_body,
        grid=(NPAD // BR,),
        in_specs=[pl.BlockSpec((BR, D), lambda g: (g, 0)),
                  pl.BlockSpec((D, D), lambda g: (0, 0))],
        out_specs=pl.BlockSpec((BR, D), lambda g: (g, 0)),
        out_shape=jax.ShapeDtypeStruct((NPAD, D), jnp.float32),
    )(h, w)


def _step_body(s_ref, b_ref, w_ref, o_ref, *, act):
    h = s_ref[...] + b_ref[...]
    if act:
        h = jnp.tanh(h)
    o_ref[...] = jnp.dot(h, w_ref[...], preferred_element_type=jnp.float32)


def _step(scat, b2, w, act):
    return pl.pallas_call(
        functools.partial(_step_body, act=act),
        grid=(NPAD // BR,),
        in_specs=[pl.BlockSpec((BR, D), lambda g: (g, 0)),
                  pl.BlockSpec((1, D), lambda g: (0, 0)),
                  pl.BlockSpec((D, D), lambda g: (0, 0))],
        out_specs=pl.BlockSpec((BR, D), lambda g: (g, 0)),
        out_shape=jax.ShapeDtypeStruct((NPAD, D), jnp.float32),
    )(scat, b2, w)


_PBLK = 1000
_PNB = N // _PBLK


def _final_body(s_ref, b_ref, bt_ref, *fc_refs_and_out):
    fw = fc_refs_and_out[:6]
    fb = fc_refs_and_out[6:12]
    o_ref = fc_refs_and_out[12]
    psum, cnt = fc_refs_and_out[13], fc_refs_and_out[14]
    g = pl.program_id(0)

    @pl.when(g == 0)
    def _():
        psum[...] = jnp.zeros((G, D), jnp.float32)
        cnt[...] = jnp.zeros((G, D), jnp.float32)

    h6 = jnp.tanh(s_ref[...] + b_ref[...])
    bt = bt_ref[...].reshape(1, _PBLK)
    oh = (bt == lax.broadcasted_iota(jnp.int32, (G, 1), 0)).astype(jnp.float32)
    psum[...] += jnp.dot(oh, h6, preferred_element_type=jnp.float32)
    cnt[...] += jnp.broadcast_to(
        jnp.sum(oh, axis=1, keepdims=True), (G, D))

    @pl.when(g == _PNB - 1)
    def _():
        p = psum[...] / jnp.maximum(cnt[...], 1.0)
        for i in range(6):
            p = jnp.maximum(
                jnp.dot(p, fw[i][...], preferred_element_type=jnp.float32)
                + fb[i][...], 0.0)
        o_ref[...] = p


def _final(scat, b2, batch2, fws, fbs):
    return pl.pallas_call(
        _final_body,
        grid=(_PNB,),
        in_specs=[pl.BlockSpec((_PBLK, D), lambda g: (g, 0)),
                  pl.BlockSpec((1, D), lambda g: (0, 0)),
                  pl.BlockSpec((1, 1, _PBLK), lambda g: (g, 0, 0))]
                 + [pl.BlockSpec((D, D), lambda g: (0, 0))] * 6
                 + [pl.BlockSpec((1, D), lambda g: (0, 0))] * 6,
        out_specs=pl.BlockSpec((G, D), lambda g: (0, 0)),
        out_shape=jax.ShapeDtypeStruct((G, D), jnp.float32),
        scratch_shapes=[pltpu.VMEM((G, D), jnp.float32),
                        pltpu.VMEM((G, D), jnp.float32)],
    )(scat, b2, batch2, *fws, *fbs)


def kernel(x, edge_index, edge_attr, batch, params):
    xpad = jnp.concatenate(
        [x, jnp.zeros((NPAD - N, D), jnp.float32)], axis=0)

    pk, wgt, cnts, deg = _bucket(edge_index[0], edge_index[1], edge_attr)
    dinv = _dinv(deg)

    xw = _mm0(xpad, params["conv_W0"])
    scat = None
    for i in range(6):
        scat = _scatter(xw, dinv, pk, wgt, cnts)
        if i < 5:
            b2 = params[f"conv_b{i}"].reshape(1, D)
            xw = _step(scat, b2, params[f"conv_W{i + 1}"], act=(i != 2))

    b52 = params["conv_b5"].reshape(1, D)
    batch2 = batch.reshape(_PNB, 1, _PBLK)
    fws = [params[f"fc_W{i}"] for i in range(6)]
    fbs = [params[f"fc_b{i}"].reshape(1, D) for i in range(6)]
    return _final(scat, b52, batch2, fws, fbs)
